# baseline (speedup 1.0000x reference)
import functools

import jax
import jax.numpy as jnp
from jax import lax
from jax.experimental import pallas as pl
from jax.experimental.pallas import tpu as pltpu
from jax.experimental.pallas import tpu_sc as plsc

_H = 64
_EMB = 64
_G = 8
_NEIGH = 4.0
_NPED = 512
_NSEQ = 8
_GSN = _NEIGH / (_G - 1)

_NC = 2
_NS = 16
_NW = _NC * _NS
_IPW = _NPED // _NW
_CH = 128
_PROWS = 80


def _sc_pool(hid_hbm, posT_hbm, seq_hbm, pooled_hbm,
             hid_v, px_v, py_v, seq_v, idx_v, zero_v, pacc_sh, sem):
    G = _G
    half = G // 2
    cid = lax.axis_index("c")
    sid = lax.axis_index("s")
    wid = sid * _NC + cid

    pltpu.sync_copy(hid_hbm, hid_v)
    pltpu.sync_copy(posT_hbm.at[0], px_v)
    pltpu.sync_copy(posT_hbm.at[1], py_v)
    pltpu.sync_copy(seq_hbm, seq_v)

    lanes = lax.broadcasted_iota(jnp.int32, (16,), 0)
    seqvec = seq_v[pl.ds(0, 16)]

    def _lane_i32(vec, lane):
        return lax.reduce_sum_p.bind(
            jnp.where(lanes == lane, vec, 0), axes=(0,))

    sts = [_lane_i32(seqvec, 2 * s) for s in range(_NSEQ)]
    ens = [_lane_i32(seqvec, 2 * s + 1) for s in range(_NSEQ)]

    zv = jnp.zeros((16,), jnp.float32)
    for r in range(_PROWS // 4):
        for k in range(_H // 16):
            zero_v[r, pl.ds(16 * k, 16)] = zv

    myacc = pacc_sh.at[sid]

    def per_i(ii, _):
        i = wid + _NW * ii

        for q in range(4):
            pltpu.sync_copy(
                zero_v, myacc.at[pl.ds(q * (_PROWS // 4), _PROWS // 4)])

        ib = 16 * (i // 16)
        il = i - ib
        pxi = lax.reduce_sum_p.bind(
            jnp.where(lanes == il, px_v[pl.ds(ib, 16)], 0.0), axes=(0,))
        pyi = lax.reduce_sum_p.bind(
            jnp.where(lanes == il, py_v[pl.ds(ib, 16)], 0.0), axes=(0,))

        for s in range(_NSEQ):
            st = sts[s]
            en = ens[s]
            isin = (i >= st) & (i < en)

            @pl.when(isin)
            def _seq_pass():
                nch = (en - st + (_CH - 1)) // _CH

                def per_chunk(c, _):
                    jb = jnp.minimum(st + c * _CH, _NPED - _CH)
                    for v in range(_CH // 16):
                        jj = jb + 16 * v + lanes
                        fx = jnp.clip((px_v[pl.ds(jb + 16 * v, 16)] - pxi)
                                      / _GSN, -half, half) + half
                        fy = jnp.clip((py_v[pl.ds(jb + 16 * v, 16)] - pyi)
                                      / _GSN, -half, half) + half
                        gx = fx.astype(jnp.int32)
                        gy = fy.astype(jnp.int32)
                        ok = ((gx < G) & (gy < G) & (jj != i)
                              & (jj >= st) & (jj < en))
                        idx_v[0, pl.ds(16 * v, 16)] = jnp.where(
                            ok, gy * G + gx, G * G + (jj & 15))
                    pltpu.sync_copy(hid_v.at[pl.ds(jb, _CH)],
                                    myacc.at[idx_v.at[0]], add=True)
                    return 0

                lax.fori_loop(0, nch, per_chunk, 0)

        pltpu.sync_copy(myacc.at[pl.ds(0, G * G)], pooled_hbm.at[i])
        return 0

    lax.fori_loop(0, _IPW, per_i, 0)


def _linear_kernel(pooled_ref, wt_ref, b_ref, out_ref):
    acc = lax.dot_general(
        pooled_ref[...], wt_ref[...], (((1,), (0,)), ((), ())),
        preferred_element_type=jnp.float32)
    out_ref[...] = jnp.maximum(acc + b_ref[0:1, :], 0.0)


def kernel(hidden_states, seq_start_end, curr_pos, W, b):
    posT = curr_pos.T
    seq = seq_start_end.astype(jnp.int32).reshape(16)

    mesh = plsc.VectorSubcoreMesh(core_axis_name="c", subcore_axis_name="s")
    pooled = pl.kernel(
        _sc_pool,
        out_type=jax.ShapeDtypeStruct((_NPED, _G * _G, _H), jnp.float32),
        mesh=mesh,
        compiler_params=pltpu.CompilerParams(
            needs_layout_passes=False, use_tc_tiling_on_sc=False),
        scratch_types=[
            pltpu.VMEM((_NPED, _H), jnp.float32),
            pltpu.VMEM((_NPED,), jnp.float32),
            pltpu.VMEM((_NPED,), jnp.float32),
            pltpu.VMEM((16,), jnp.int32),
            pltpu.VMEM((2, _CH), jnp.int32),
            pltpu.VMEM((_PROWS // 4, _H), jnp.float32),
            pltpu.VMEM_SHARED((_NS, _PROWS, _H), jnp.float32),
            pltpu.SemaphoreType.DMA,
        ],
    )(hidden_states, posT, seq)

    return pl.pallas_call(
        _linear_kernel,
        out_shape=jax.ShapeDtypeStruct((_NPED, _EMB), jnp.float32),
        in_specs=[
            pl.BlockSpec(memory_space=pltpu.VMEM),
            pl.BlockSpec(memory_space=pltpu.VMEM),
            pl.BlockSpec(memory_space=pltpu.VMEM),
        ],
        out_specs=pl.BlockSpec(memory_space=pltpu.VMEM),
    )(pooled.reshape(_NPED, _G * _G * _H), W.T, b.reshape(1, _EMB))


# speedup vs baseline: 1.0062x; 1.0062x over previous
import functools

import jax
import jax.numpy as jnp
from jax import lax
from jax.experimental import pallas as pl
from jax.experimental.pallas import tpu as pltpu
from jax.experimental.pallas import tpu_sc as plsc

_H = 64
_EMB = 64
_G = 8
_NEIGH = 4.0
_NPED = 512
_NSEQ = 8
_GSN = _NEIGH / (_G - 1)

_NC = 2
_NS = 16
_NW = _NC * _NS
_IPW = _NPED // _NW
_CH = 128
_PROWS = 80


def _sc_pool(hid_hbm, posT_hbm, seq_hbm, pooled_hbm,
             hid_v, px_v, py_v, seq_v, idx_v, zero_v, pacc_sh, sem):
    G = _G
    half = G // 2
    cid = lax.axis_index("c")
    sid = lax.axis_index("s")
    wid = sid * _NC + cid

    pltpu.sync_copy(hid_hbm, hid_v)
    pltpu.sync_copy(posT_hbm.at[0], px_v)
    pltpu.sync_copy(posT_hbm.at[1], py_v)
    pltpu.sync_copy(seq_hbm, seq_v)

    lanes = lax.broadcasted_iota(jnp.int32, (16,), 0)
    seqvec = seq_v[pl.ds(0, 16)]

    def _lane_i32(vec, lane):
        return lax.reduce_sum_p.bind(
            jnp.where(lanes == lane, vec, 0), axes=(0,))

    sts = [_lane_i32(seqvec, 2 * s) for s in range(_NSEQ)]
    ens = [_lane_i32(seqvec, 2 * s + 1) for s in range(_NSEQ)]

    zv = jnp.zeros((16,), jnp.float32)
    for r in range(_PROWS // 4):
        for k in range(_H // 16):
            zero_v[r, pl.ds(16 * k, 16)] = zv

    myacc = pacc_sh.at[sid]

    def per_i(ii, _):
        i = wid + _NW * ii

        for q in range(4):
            pltpu.sync_copy(
                zero_v, myacc.at[pl.ds(q * (_PROWS // 4), _PROWS // 4)])

        ib = 16 * (i // 16)
        il = i - ib
        pxi = lax.reduce_sum_p.bind(
            jnp.where(lanes == il, px_v[pl.ds(ib, 16)], 0.0), axes=(0,))
        pyi = lax.reduce_sum_p.bind(
            jnp.where(lanes == il, py_v[pl.ds(ib, 16)], 0.0), axes=(0,))

        for s in range(_NSEQ):
            st = sts[s]
            en = ens[s]
            isin = (i >= st) & (i < en)

            @pl.when(isin)
            def _seq_pass():
                nch = (en - st + (_CH - 1)) // _CH

                def per_chunk(c, _):
                    lo_c = st + c * _CH
                    hi_c = jnp.minimum(lo_c + _CH, en)
                    jb = jnp.minimum(lo_c, _NPED - _CH)
                    for v in range(_CH // 16):
                        jj = jb + 16 * v + lanes
                        fx = jnp.clip((px_v[pl.ds(jb + 16 * v, 16)] - pxi)
                                      / _GSN, -half, half) + half
                        fy = jnp.clip((py_v[pl.ds(jb + 16 * v, 16)] - pyi)
                                      / _GSN, -half, half) + half
                        gx = fx.astype(jnp.int32)
                        gy = fy.astype(jnp.int32)
                        ok = ((gx < G) & (gy < G) & (jj != i)
                              & (jj >= lo_c) & (jj < hi_c))
                        idx_v[0, pl.ds(16 * v, 16)] = jnp.where(
                            ok, gy * G + gx, G * G + (jj & 15))
                    pltpu.sync_copy(hid_v.at[pl.ds(jb, _CH)],
                                    myacc.at[idx_v.at[0]], add=True)
                    return 0

                lax.fori_loop(0, nch, per_chunk, 0)

        pltpu.sync_copy(myacc.at[pl.ds(0, G * G)], pooled_hbm.at[i])
        return 0

    lax.fori_loop(0, _IPW, per_i, 0)


def _linear_kernel(pooled_ref, wt_ref, b_ref, out_ref):
    acc = lax.dot_general(
        pooled_ref[...], wt_ref[...], (((1,), (0,)), ((), ())),
        preferred_element_type=jnp.float32)
    out_ref[...] = jnp.maximum(acc + b_ref[0:1, :], 0.0)


def kernel(hidden_states, seq_start_end, curr_pos, W, b):
    posT = curr_pos.T
    seq = seq_start_end.astype(jnp.int32).reshape(16)

    mesh = plsc.VectorSubcoreMesh(core_axis_name="c", subcore_axis_name="s")
    pooled = pl.kernel(
        _sc_pool,
        out_type=jax.ShapeDtypeStruct((_NPED, _G * _G, _H), jnp.float32),
        mesh=mesh,
        compiler_params=pltpu.CompilerParams(
            needs_layout_passes=False, use_tc_tiling_on_sc=False),
        scratch_types=[
            pltpu.VMEM((_NPED, _H), jnp.float32),
            pltpu.VMEM((_NPED,), jnp.float32),
            pltpu.VMEM((_NPED,), jnp.float32),
            pltpu.VMEM((16,), jnp.int32),
            pltpu.VMEM((2, _CH), jnp.int32),
            pltpu.VMEM((_PROWS // 4, _H), jnp.float32),
            pltpu.VMEM_SHARED((_NS, _PROWS, _H), jnp.float32),
            pltpu.SemaphoreType.DMA,
        ],
    )(hidden_states, posT, seq)

    return pl.pallas_call(
        _linear_kernel,
        out_shape=jax.ShapeDtypeStruct((_NPED, _EMB), jnp.float32),
        in_specs=[
            pl.BlockSpec(memory_space=pltpu.VMEM),
            pl.BlockSpec(memory_space=pltpu.VMEM),
            pl.BlockSpec(memory_space=pltpu.VMEM),
        ],
        out_specs=pl.BlockSpec(memory_space=pltpu.VMEM),
    )(pooled.reshape(_NPED, _G * _G * _H), W.T, b.reshape(1, _EMB))


# speedup vs baseline: 1.1050x; 1.0982x over previous
import functools

import jax
import jax.numpy as jnp
from jax import lax
from jax.experimental import pallas as pl
from jax.experimental.pallas import tpu as pltpu
from jax.experimental.pallas import tpu_sc as plsc

_H = 64
_EMB = 64
_G = 8
_NEIGH = 4.0
_NPED = 512
_NSEQ = 8
_GSN = _NEIGH / (_G - 1)

_NC = 2
_NS = 16
_NW = _NC * _NS
_IPW = _NPED // _NW
_CH = 128
_PROWS = 80


def _sc_pool(hid_hbm, posT_hbm, seq_hbm, pooled_hbm,
             hid_v, px_v, py_v, seq_v, idx_v, zero_v, pacc_sh, sem, sem_o):
    G = _G
    half = G // 2
    cid = lax.axis_index("c")
    sid = lax.axis_index("s")
    wid = sid * _NC + cid

    pltpu.sync_copy(hid_hbm, hid_v)
    pltpu.sync_copy(posT_hbm.at[0], px_v)
    pltpu.sync_copy(posT_hbm.at[1], py_v)
    pltpu.sync_copy(seq_hbm, seq_v)

    lanes = lax.broadcasted_iota(jnp.int32, (16,), 0)
    seqvec = seq_v[pl.ds(0, 16)]

    def _lane_i32(vec, lane):
        return lax.reduce_sum_p.bind(
            jnp.where(lanes == lane, vec, 0), axes=(0,))

    sts = [_lane_i32(seqvec, 2 * s) for s in range(_NSEQ)]
    ens = [_lane_i32(seqvec, 2 * s + 1) for s in range(_NSEQ)]

    zv = jnp.zeros((16,), jnp.float32)
    for r in range(_PROWS):
        for k in range(_H // 16):
            zero_v[r, pl.ds(16 * k, 16)] = zv

    myslabs = pacc_sh.at[sid]

    zdescs = [pltpu.async_copy(zero_v, myslabs.at[q], sem_o)
              for q in range(_IPW)]
    for d in zdescs:
        d.wait()

    def per_i(ii, _):
        i = wid + _NW * ii
        myacc = myslabs.at[ii]

        ib = 16 * (i // 16)
        il = i - ib
        pxi = lax.reduce_sum_p.bind(
            jnp.where(lanes == il, px_v[pl.ds(ib, 16)], 0.0), axes=(0,))
        pyi = lax.reduce_sum_p.bind(
            jnp.where(lanes == il, py_v[pl.ds(ib, 16)], 0.0), axes=(0,))

        for s in range(_NSEQ):
            st = sts[s]
            en = ens[s]
            isin = (i >= st) & (i < en)

            @pl.when(isin)
            def _seq_pass():
                nch = (en - st + (_CH - 1)) // _CH

                def per_chunk(c, _):
                    lo_c = st + c * _CH
                    hi_c = jnp.minimum(lo_c + _CH, en)
                    jb = jnp.minimum(lo_c, _NPED - _CH)
                    for v in range(_CH // 16):
                        jj = jb + 16 * v + lanes
                        fx = jnp.clip((px_v[pl.ds(jb + 16 * v, 16)] - pxi)
                                      / _GSN, -half, half) + half
                        fy = jnp.clip((py_v[pl.ds(jb + 16 * v, 16)] - pyi)
                                      / _GSN, -half, half) + half
                        gx = fx.astype(jnp.int32)
                        gy = fy.astype(jnp.int32)
                        ok = ((gx < G) & (gy < G) & (jj != i)
                              & (jj >= lo_c) & (jj < hi_c))
                        idx_v[0, pl.ds(16 * v, 16)] = jnp.where(
                            ok, gy * G + gx, G * G + (jj & 15))
                    pltpu.sync_copy(hid_v.at[pl.ds(jb, _CH)],
                                    myacc.at[idx_v.at[0]], add=True)
                    return 0

                lax.fori_loop(0, nch, per_chunk, 0)

        pltpu.async_copy(myacc.at[pl.ds(0, G * G)], pooled_hbm.at[i], sem_o)
        return 0

    lax.fori_loop(0, _IPW, per_i, 0)

    for q in range(_IPW):
        pltpu.make_async_copy(
            myslabs.at[0].at[pl.ds(0, _G * _G)], pooled_hbm.at[0], sem_o
        ).wait()


def _linear_kernel(pooled_ref, wt_ref, b_ref, out_ref):
    acc = lax.dot_general(
        pooled_ref[...], wt_ref[...], (((1,), (0,)), ((), ())),
        preferred_element_type=jnp.float32)
    out_ref[...] = jnp.maximum(acc + b_ref[0:1, :], 0.0)


def kernel(hidden_states, seq_start_end, curr_pos, W, b):
    posT = curr_pos.T
    seq = seq_start_end.astype(jnp.int32).reshape(16)

    mesh = plsc.VectorSubcoreMesh(core_axis_name="c", subcore_axis_name="s")
    pooled = pl.kernel(
        _sc_pool,
        out_type=jax.ShapeDtypeStruct((_NPED, _G * _G, _H), jnp.float32),
        mesh=mesh,
        compiler_params=pltpu.CompilerParams(
            needs_layout_passes=False, use_tc_tiling_on_sc=False),
        scratch_types=[
            pltpu.VMEM((_NPED, _H), jnp.float32),
            pltpu.VMEM((_NPED,), jnp.float32),
            pltpu.VMEM((_NPED,), jnp.float32),
            pltpu.VMEM((16,), jnp.int32),
            pltpu.VMEM((2, _CH), jnp.int32),
            pltpu.VMEM((_PROWS, _H), jnp.float32),
            pltpu.VMEM_SHARED((_NS, _IPW, _PROWS, _H), jnp.float32),
            pltpu.SemaphoreType.DMA,
            pltpu.SemaphoreType.DMA,
        ],
    )(hidden_states, posT, seq)

    return pl.pallas_call(
        _linear_kernel,
        out_shape=jax.ShapeDtypeStruct((_NPED, _EMB), jnp.float32),
        in_specs=[
            pl.BlockSpec(memory_space=pltpu.VMEM),
            pl.BlockSpec(memory_space=pltpu.VMEM),
            pl.BlockSpec(memory_space=pltpu.VMEM),
        ],
        out_specs=pl.BlockSpec(memory_space=pltpu.VMEM),
    )(pooled.reshape(_NPED, _G * _G * _H), W.T, b.reshape(1, _EMB))


# speedup vs baseline: 1.1923x; 1.0790x over previous
import functools

import jax
import jax.numpy as jnp
from jax import lax
from jax.experimental import pallas as pl
from jax.experimental.pallas import tpu as pltpu
from jax.experimental.pallas import tpu_sc as plsc

_H = 64
_EMB = 64
_G = 8
_NEIGH = 4.0
_NPED = 512
_NSEQ = 8
_GSN = _NEIGH / (_G - 1)

_NC = 2
_NS = 16
_NW = _NC * _NS
_IPW = _NPED // _NW
_CH = 128
_PROWS = 80


def _sc_pool(hid_hbm, posT_hbm, seq_hbm, pooled_hbm,
             hid_v, px_v, py_v, seq_v, idx_v, zero_v, pacc_sh, sem, sem_o):
    G = _G
    half = G // 2
    cid = lax.axis_index("c")
    sid = lax.axis_index("s")
    wid = sid * _NC + cid

    pltpu.sync_copy(hid_hbm, hid_v)
    pltpu.sync_copy(posT_hbm.at[0], px_v)
    pltpu.sync_copy(posT_hbm.at[1], py_v)
    pltpu.sync_copy(seq_hbm, seq_v)

    lanes = lax.broadcasted_iota(jnp.int32, (16,), 0)
    seqvec = seq_v[pl.ds(0, 16)]

    def _lane_i32(vec, lane):
        return lax.reduce_sum_p.bind(
            jnp.where(lanes == lane, vec, 0), axes=(0,))

    sts = [_lane_i32(seqvec, 2 * s) for s in range(_NSEQ)]
    ens = [_lane_i32(seqvec, 2 * s + 1) for s in range(_NSEQ)]

    zv = jnp.zeros((16,), jnp.float32)
    for r in range(_PROWS):
        for k in range(_H // 16):
            zero_v[r, pl.ds(16 * k, 16)] = zv

    myslabs = pacc_sh.at[sid]

    zdescs = [pltpu.async_copy(zero_v, myslabs.at[q], sem_o)
              for q in range(_IPW)]
    for d in zdescs:
        d.wait()

    def per_i(ii, _):
        i = wid + _NW * ii
        myacc = myslabs.at[ii]

        ib = 16 * (i // 16)
        il = i - ib
        pxi = lax.reduce_sum_p.bind(
            jnp.where(lanes == il, px_v[pl.ds(ib, 16)], 0.0), axes=(0,))
        pyi = lax.reduce_sum_p.bind(
            jnp.where(lanes == il, py_v[pl.ds(ib, 16)], 0.0), axes=(0,))

        cnt = jnp.int32(0)
        for s in range(_NSEQ):
            st = sts[s]
            en = ens[s]
            isin = (i >= st) & (i < en)

            def _seq_pass(cnt, st=st, en=en):
                nch = (en - st + (_CH - 1)) // _CH

                def per_chunk(c, cnt):
                    slot = cnt % 2

                    @pl.when(cnt >= 2)
                    def _():
                        pltpu.make_async_copy(
                            hid_v.at[pl.ds(0, _CH)],
                            myacc.at[idx_v.at[slot]], sem).wait()

                    lo_c = st + c * _CH
                    hi_c = jnp.minimum(lo_c + _CH, en)
                    jb = jnp.minimum(lo_c, _NPED - _CH)
                    for v in range(_CH // 16):
                        jj = jb + 16 * v + lanes
                        fx = jnp.clip((px_v[pl.ds(jb + 16 * v, 16)] - pxi)
                                      / _GSN, -half, half) + half
                        fy = jnp.clip((py_v[pl.ds(jb + 16 * v, 16)] - pyi)
                                      / _GSN, -half, half) + half
                        gx = fx.astype(jnp.int32)
                        gy = fy.astype(jnp.int32)
                        ok = ((gx < G) & (gy < G) & (jj != i)
                              & (jj >= lo_c) & (jj < hi_c))
                        idx_v[slot, pl.ds(16 * v, 16)] = jnp.where(
                            ok, gy * G + gx, G * G + (jj & 15))
                    pltpu.async_copy(hid_v.at[pl.ds(jb, _CH)],
                                     myacc.at[idx_v.at[slot]], sem, add=True)
                    return cnt + 1

                return lax.fori_loop(0, nch, per_chunk, cnt)

            cnt = lax.cond(isin, _seq_pass, lambda c: c, cnt)

        @pl.when(cnt >= 1)
        def _():
            pltpu.make_async_copy(
                hid_v.at[pl.ds(0, _CH)], myacc.at[idx_v.at[0]], sem).wait()

        @pl.when(cnt >= 2)
        def _():
            pltpu.make_async_copy(
                hid_v.at[pl.ds(0, _CH)], myacc.at[idx_v.at[1]], sem).wait()

        pltpu.async_copy(myacc.at[pl.ds(0, G * G)], pooled_hbm.at[i], sem_o)
        return 0

    lax.fori_loop(0, _IPW, per_i, 0)

    for q in range(_IPW):
        pltpu.make_async_copy(
            myslabs.at[0].at[pl.ds(0, _G * _G)], pooled_hbm.at[0], sem_o
        ).wait()


def _linear_kernel(pooled_ref, wt_ref, b_ref, out_ref):
    acc = lax.dot_general(
        pooled_ref[...], wt_ref[...], (((1,), (0,)), ((), ())),
        preferred_element_type=jnp.float32)
    out_ref[...] = jnp.maximum(acc + b_ref[0:1, :], 0.0)


def kernel(hidden_states, seq_start_end, curr_pos, W, b):
    posT = curr_pos.T
    seq = seq_start_end.astype(jnp.int32).reshape(16)

    mesh = plsc.VectorSubcoreMesh(core_axis_name="c", subcore_axis_name="s")
    pooled = pl.kernel(
        _sc_pool,
        out_type=jax.ShapeDtypeStruct((_NPED, _G * _G, _H), jnp.float32),
        mesh=mesh,
        compiler_params=pltpu.CompilerParams(
            needs_layout_passes=False, use_tc_tiling_on_sc=False),
        scratch_types=[
            pltpu.VMEM((_NPED, _H), jnp.float32),
            pltpu.VMEM((_NPED,), jnp.float32),
            pltpu.VMEM((_NPED,), jnp.float32),
            pltpu.VMEM((16,), jnp.int32),
            pltpu.VMEM((2, _CH), jnp.int32),
            pltpu.VMEM((_PROWS, _H), jnp.float32),
            pltpu.VMEM_SHARED((_NS, _IPW, _PROWS, _H), jnp.float32),
            pltpu.SemaphoreType.DMA,
            pltpu.SemaphoreType.DMA,
        ],
    )(hidden_states, posT, seq)

    return pl.pallas_call(
        _linear_kernel,
        out_shape=jax.ShapeDtypeStruct((_NPED, _EMB), jnp.float32),
        in_specs=[
            pl.BlockSpec(memory_space=pltpu.VMEM),
            pl.BlockSpec(memory_space=pltpu.VMEM),
            pl.BlockSpec(memory_space=pltpu.VMEM),
        ],
        out_specs=pl.BlockSpec(memory_space=pltpu.VMEM),
    )(pooled.reshape(_NPED, _G * _G * _H), W.T, b.reshape(1, _EMB))


# speedup vs baseline: 1.3493x; 1.1318x over previous
import functools

import jax
import jax.numpy as jnp
from jax import lax
from jax.experimental import pallas as pl
from jax.experimental.pallas import tpu as pltpu
from jax.experimental.pallas import tpu_sc as plsc

_H = 64
_EMB = 64
_G = 8
_NEIGH = 4.0
_NPED = 512
_NSEQ = 8
_GSN = _NEIGH / (_G - 1)

_NC = 2
_NS = 16
_NW = _NC * _NS
_IPW = _NPED // _NW
_CH = 128
_PROWS = 80


def _sc_pool(hid_hbm, posT_hbm, seq_hbm, pooled_hbm,
             hid_v, px_v, py_v, seq_v, idx_v, zero_v, pacc_sh, sem, sem_o):
    G = _G
    half = G // 2
    cid = lax.axis_index("c")
    sid = lax.axis_index("s")
    wid = sid * _NC + cid

    pltpu.sync_copy(hid_hbm, hid_v)
    pltpu.sync_copy(posT_hbm.at[0], px_v)
    pltpu.sync_copy(posT_hbm.at[1], py_v)
    pltpu.sync_copy(seq_hbm, seq_v)

    lanes = lax.broadcasted_iota(jnp.int32, (16,), 0)
    seqvec = seq_v[pl.ds(0, 16)]

    def _lane_i32(vec, lane):
        return lax.reduce_sum_p.bind(
            jnp.where(lanes == lane, vec, 0), axes=(0,))

    sts = [_lane_i32(seqvec, 2 * s) for s in range(_NSEQ)]
    ens = [_lane_i32(seqvec, 2 * s + 1) for s in range(_NSEQ)]

    zv = jnp.zeros((16,), jnp.float32)
    for r in range(_PROWS):
        for k in range(_H // 16):
            zero_v[r, pl.ds(16 * k, 16)] = zv

    myslabs = pacc_sh.at[sid]

    zdescs = [pltpu.async_copy(zero_v, myslabs.at[q], sem_o)
              for q in range(_IPW)]
    for d in zdescs:
        d.wait()

    def per_i(ii, _):
        i = wid + _NW * ii
        myacc = myslabs.at[ii]

        ib = 16 * (i // 16)
        il = i - ib
        pxi = lax.reduce_sum_p.bind(
            jnp.where(lanes == il, px_v[pl.ds(ib, 16)], 0.0), axes=(0,))
        pyi = lax.reduce_sum_p.bind(
            jnp.where(lanes == il, py_v[pl.ds(ib, 16)], 0.0), axes=(0,))

        cnt = jnp.int32(0)
        for s in range(_NSEQ):
            st = sts[s]
            en = ens[s]
            isin = (i >= st) & (i < en)

            def _seq_pass(cnt, st=st, en=en):
                nch = (en - st + (_CH - 1)) // _CH

                def per_chunk(c, cnt):
                    slot = cnt % 2

                    @pl.when(cnt >= 2)
                    def _():
                        pltpu.make_async_copy(
                            hid_v.at[pl.ds(0, _CH)],
                            myacc.at[idx_v.at[slot]], sem).wait()

                    lo_c = st + c * _CH
                    hi_c = jnp.minimum(lo_c + _CH, en)
                    jb = jnp.minimum(lo_c, _NPED - _CH)
                    for v in range(_CH // 16):
                        jj = jb + 16 * v + lanes
                        fx = jnp.clip((px_v[pl.ds(jb + 16 * v, 16)] - pxi)
                                      / _GSN, -half, half) + half
                        fy = jnp.clip((py_v[pl.ds(jb + 16 * v, 16)] - pyi)
                                      / _GSN, -half, half) + half
                        gx = fx.astype(jnp.int32)
                        gy = fy.astype(jnp.int32)
                        ok = ((gx < G) & (gy < G) & (jj != i)
                              & (jj >= lo_c) & (jj < hi_c))
                        idx_v[slot, pl.ds(16 * v, 16)] = jnp.where(
                            ok, gy * G + gx, G * G + (jj & 15))
                    pltpu.async_copy(hid_v.at[pl.ds(jb, _CH)],
                                     myacc.at[idx_v.at[slot]], sem, add=True)
                    return cnt + 1

                return lax.fori_loop(0, nch, per_chunk, cnt)

            cnt = lax.cond(isin, _seq_pass, lambda c: c, cnt)

        @pl.when(cnt >= 1)
        def _():
            pltpu.make_async_copy(
                hid_v.at[pl.ds(0, _CH)], myacc.at[idx_v.at[0]], sem).wait()

        @pl.when(cnt >= 2)
        def _():
            pltpu.make_async_copy(
                hid_v.at[pl.ds(0, _CH)], myacc.at[idx_v.at[1]], sem).wait()

        pltpu.async_copy(myacc.at[pl.ds(0, G * G)], pooled_hbm.at[i], sem_o)
        return 0

    lax.fori_loop(0, _IPW, per_i, 0)

    for q in range(_IPW):
        pltpu.make_async_copy(
            myslabs.at[0].at[pl.ds(0, _G * _G)], pooled_hbm.at[0], sem_o
        ).wait()


def _linear_kernel(pooled_ref, wt_ref, b_ref, out_ref):
    flat = pooled_ref[...].reshape(_NPED, _G * _G * _H)
    acc = lax.dot_general(
        flat, wt_ref[...], (((1,), (0,)), ((), ())),
        preferred_element_type=jnp.float32)
    out_ref[...] = jnp.maximum(acc + b_ref[0:1, :], 0.0)


def kernel(hidden_states, seq_start_end, curr_pos, W, b):
    posT = curr_pos.T
    seq = seq_start_end.astype(jnp.int32).reshape(16)

    mesh = plsc.VectorSubcoreMesh(core_axis_name="c", subcore_axis_name="s")
    pooled = pl.kernel(
        _sc_pool,
        out_type=jax.ShapeDtypeStruct((_NPED, _G * _G, _H), jnp.float32),
        mesh=mesh,
        compiler_params=pltpu.CompilerParams(
            needs_layout_passes=False, use_tc_tiling_on_sc=False),
        scratch_types=[
            pltpu.VMEM((_NPED, _H), jnp.float32),
            pltpu.VMEM((_NPED,), jnp.float32),
            pltpu.VMEM((_NPED,), jnp.float32),
            pltpu.VMEM((16,), jnp.int32),
            pltpu.VMEM((2, _CH), jnp.int32),
            pltpu.VMEM((_PROWS, _H), jnp.float32),
            pltpu.VMEM_SHARED((_NS, _IPW, _PROWS, _H), jnp.float32),
            pltpu.SemaphoreType.DMA,
            pltpu.SemaphoreType.DMA,
        ],
    )(hidden_states, posT, seq)

    return pl.pallas_call(
        _linear_kernel,
        out_shape=jax.ShapeDtypeStruct((_NPED, _EMB), jnp.float32),
        in_specs=[
            pl.BlockSpec(memory_space=pltpu.VMEM),
            pl.BlockSpec(memory_space=pltpu.VMEM),
            pl.BlockSpec(memory_space=pltpu.VMEM),
        ],
        out_specs=pl.BlockSpec(memory_space=pltpu.VMEM),
    )(pooled, W.T, b.reshape(1, _EMB))


# speedup vs baseline: 1.3497x; 1.0002x over previous
import functools

import jax
import jax.numpy as jnp
from jax import lax
from jax.experimental import pallas as pl
from jax.experimental.pallas import tpu as pltpu
from jax.experimental.pallas import tpu_sc as plsc

_H = 64
_EMB = 64
_G = 8
_NEIGH = 4.0
_NPED = 512
_NSEQ = 8
_GSN = _NEIGH / (_G - 1)

_NC = 2
_NS = 16
_NW = _NC * _NS
_IPW = _NPED // _NW
_CH = 128
_PROWS = 80


def _sc_pool(hid_hbm, posT_hbm, seq_hbm, pooled_hbm,
             hid_v, px_v, py_v, seq_v, idx_v, zero_v, pacc_sh, sem, sem_o):
    G = _G
    half = G // 2
    cid = lax.axis_index("c")
    sid = lax.axis_index("s")
    wid = sid * _NC + cid

    pltpu.sync_copy(hid_hbm, hid_v)
    pltpu.sync_copy(posT_hbm.at[0], px_v)
    pltpu.sync_copy(posT_hbm.at[1], py_v)
    pltpu.sync_copy(seq_hbm, seq_v)

    lanes = lax.broadcasted_iota(jnp.int32, (16,), 0)
    seqvec = seq_v[pl.ds(0, 16)]

    def _lane_i32(vec, lane):
        return lax.reduce_sum_p.bind(
            jnp.where(lanes == lane, vec, 0), axes=(0,))

    sts = [_lane_i32(seqvec, 2 * s) for s in range(_NSEQ)]
    ens = [_lane_i32(seqvec, 2 * s + 1) for s in range(_NSEQ)]

    zv = jnp.zeros((16,), jnp.float32)
    for r in range(_PROWS):
        for k in range(_H // 16):
            zero_v[r, pl.ds(16 * k, 16)] = zv

    myslabs = pacc_sh.at[sid]

    zdescs = [pltpu.async_copy(zero_v, myslabs.at[q], sem_o)
              for q in range(_IPW)]
    for d in zdescs:
        d.wait()

    def per_i(ii, _):
        i = wid + _NW * ii
        myacc = myslabs.at[ii]

        ib = 16 * (i // 16)
        il = i - ib
        pxi = lax.reduce_sum_p.bind(
            jnp.where(lanes == il, px_v[pl.ds(ib, 16)], 0.0), axes=(0,))
        pyi = lax.reduce_sum_p.bind(
            jnp.where(lanes == il, py_v[pl.ds(ib, 16)], 0.0), axes=(0,))

        cnt = jnp.int32(0)
        for s in range(_NSEQ):
            st = sts[s]
            en = ens[s]
            isin = (i >= st) & (i < en)

            def _seq_pass(cnt, st=st, en=en):
                nch = (en - st + (_CH - 1)) // _CH

                def per_chunk(c, cnt):
                    slot = cnt % 4

                    @pl.when(cnt >= 4)
                    def _():
                        pltpu.make_async_copy(
                            hid_v.at[pl.ds(0, _CH)],
                            myacc.at[idx_v.at[slot]], sem).wait()

                    lo_c = st + c * _CH
                    hi_c = jnp.minimum(lo_c + _CH, en)
                    jb = jnp.minimum(lo_c, _NPED - _CH)
                    for v in range(_CH // 16):
                        jj = jb + 16 * v + lanes
                        fx = jnp.clip((px_v[pl.ds(jb + 16 * v, 16)] - pxi)
                                      / _GSN, -half, half) + half
                        fy = jnp.clip((py_v[pl.ds(jb + 16 * v, 16)] - pyi)
                                      / _GSN, -half, half) + half
                        gx = fx.astype(jnp.int32)
                        gy = fy.astype(jnp.int32)
                        ok = ((gx < G) & (gy < G) & (jj != i)
                              & (jj >= lo_c) & (jj < hi_c))
                        idx_v[slot, pl.ds(16 * v, 16)] = jnp.where(
                            ok, gy * G + gx, G * G + (jj & 15))
                    pltpu.async_copy(hid_v.at[pl.ds(jb, _CH)],
                                     myacc.at[idx_v.at[slot]], sem, add=True)
                    return cnt + 1

                return lax.fori_loop(0, nch, per_chunk, cnt)

            cnt = lax.cond(isin, _seq_pass, lambda c: c, cnt)

        for q in range(4):
            @pl.when(cnt >= q + 1)
            def _(q=q):
                pltpu.make_async_copy(
                    hid_v.at[pl.ds(0, _CH)], myacc.at[idx_v.at[q]], sem).wait()

        pltpu.async_copy(myacc.at[pl.ds(0, G * G)], pooled_hbm.at[i], sem_o)
        return 0

    lax.fori_loop(0, _IPW, per_i, 0)

    for q in range(_IPW):
        pltpu.make_async_copy(
            myslabs.at[0].at[pl.ds(0, _G * _G)], pooled_hbm.at[0], sem_o
        ).wait()


def _linear_kernel(pooled_ref, wt_ref, b_ref, out_ref):
    flat = pooled_ref[...].reshape(_NPED, _G * _G * _H)
    acc = lax.dot_general(
        flat, wt_ref[...], (((1,), (0,)), ((), ())),
        preferred_element_type=jnp.float32)
    out_ref[...] = jnp.maximum(acc + b_ref[0:1, :], 0.0)


def kernel(hidden_states, seq_start_end, curr_pos, W, b):
    posT = curr_pos.T
    seq = seq_start_end.astype(jnp.int32).reshape(16)

    mesh = plsc.VectorSubcoreMesh(core_axis_name="c", subcore_axis_name="s")
    pooled = pl.kernel(
        _sc_pool,
        out_type=jax.ShapeDtypeStruct((_NPED, _G * _G, _H), jnp.float32),
        mesh=mesh,
        compiler_params=pltpu.CompilerParams(
            needs_layout_passes=False, use_tc_tiling_on_sc=False),
        scratch_types=[
            pltpu.VMEM((_NPED, _H), jnp.float32),
            pltpu.VMEM((_NPED,), jnp.float32),
            pltpu.VMEM((_NPED,), jnp.float32),
            pltpu.VMEM((16,), jnp.int32),
            pltpu.VMEM((4, _CH), jnp.int32),
            pltpu.VMEM((_PROWS, _H), jnp.float32),
            pltpu.VMEM_SHARED((_NS, _IPW, _PROWS, _H), jnp.float32),
            pltpu.SemaphoreType.DMA,
            pltpu.SemaphoreType.DMA,
        ],
    )(hidden_states, posT, seq)

    return pl.pallas_call(
        _linear_kernel,
        out_shape=jax.ShapeDtypeStruct((_NPED, _EMB), jnp.float32),
        in_specs=[
            pl.BlockSpec(memory_space=pltpu.VMEM),
            pl.BlockSpec(memory_space=pltpu.VMEM),
            pl.BlockSpec(memory_space=pltpu.VMEM),
        ],
        out_specs=pl.BlockSpec(memory_space=pltpu.VMEM),
    )(pooled, W.T, b.reshape(1, _EMB))
